# Initial kernel scaffold; baseline (speedup 1.0000x reference)
import jax
import jax.numpy as jnp
from jax.experimental import pallas as pl


def kernel(x, adj, W, b):
    raise NotImplementedError("write your pallas kernel here")


# speedup vs baseline: 15.2832x; 15.2832x over previous
import functools

import jax
import jax.numpy as jnp
from jax import lax
from jax.experimental import pallas as pl
from jax.experimental.pallas import tpu as pltpu, tpu_sc as plsc

N_NODES = 10000
N_EDGES = 320000
D = 128

NC = 2
NS = 16
NW = NC * NS

BLK = 128
BPT = 79
EP = NW * BPT * BLK
NBLK = EP // BLK
BPC = NBLK // NC

NT = 10112
RPT = NT // NS
PAD_ROW = N_NODES

_MESH = plsc.VectorSubcoreMesh(core_axis_name="c", subcore_axis_name="s")


def _deg_body(dstb_hbm, zeros_hbm, ones_hbm, degp_hbm,
              dst_v, ones_v, acc_sp, sem):
    c = lax.axis_index("c")
    s = lax.axis_index("s")
    pltpu.sync_copy(zeros_hbm, acc_sp.at[pl.ds(s * RPT, RPT)])
    pltpu.sync_copy(ones_hbm, ones_v)
    pltpu.sync_copy(dstb_hbm.at[c * NS + s], dst_v)
    plsc.subcore_barrier()

    def blk(j, carry):
        pltpu.sync_copy(ones_v, acc_sp.at[dst_v.at[j]], add=True)
        return carry

    lax.fori_loop(0, BPT, blk, 0)
    plsc.subcore_barrier()
    pltpu.sync_copy(acc_sp.at[pl.ds(s * RPT, RPT)],
                    degp_hbm.at[pl.ds(c * NT + s * RPT, RPT)])


def _deg_call(dstb, zeros_rt16, ones_b16):
    k = pl.kernel(
        _deg_body,
        out_type=jax.ShapeDtypeStruct((NC * NT, 16), jnp.float32),
        mesh=_MESH,
        scratch_types=[
            pltpu.VMEM((BPT, BLK), jnp.int32),
            pltpu.VMEM((BLK, 16), jnp.float32),
            pltpu.VMEM_SHARED((NT, 16), jnp.float32),
            pltpu.SemaphoreType.DMA,
        ],
    )
    return k(dstb, zeros_rt16, ones_b16)


def _agg_body(g_hbm, srcb_hbm, dstb_hbm, zeros_hbm, accp_hbm,
              src_v, dst_v, gbuf, acc_sp, sem):
    c = lax.axis_index("c")
    s = lax.axis_index("s")
    pltpu.sync_copy(zeros_hbm, acc_sp.at[pl.ds(s * RPT, RPT)])
    pltpu.sync_copy(srcb_hbm.at[c * NS + s], src_v)
    pltpu.sync_copy(dstb_hbm.at[c * NS + s], dst_v)
    plsc.subcore_barrier()

    def blk(j, carry):
        pltpu.async_copy(g_hbm.at[src_v.at[j]], gbuf, sem).wait()
        pltpu.sync_copy(gbuf, acc_sp.at[dst_v.at[j]], add=True)
        return carry

    lax.fori_loop(0, BPT, blk, 0)
    plsc.subcore_barrier()
    pltpu.sync_copy(acc_sp.at[pl.ds(s * RPT, RPT)],
                    accp_hbm.at[pl.ds(c * NT + s * RPT, RPT)])


def _agg_call(g_pad, srcb, dstb, zeros_rtd):
    k = pl.kernel(
        _agg_body,
        out_type=jax.ShapeDtypeStruct((NC * NT, D), jnp.float32),
        mesh=_MESH,
        scratch_types=[
            pltpu.VMEM((BPT, BLK), jnp.int32),
            pltpu.VMEM((BPT, BLK), jnp.int32),
            pltpu.VMEM((BLK, D), jnp.float32),
            pltpu.VMEM_SHARED((NT, D), jnp.float32),
            pltpu.SemaphoreType.DMA,
        ],
    )
    return k(g_pad, srcb, dstb, zeros_rtd)


_RB = 400
_NRB = N_NODES // _RB


def _gw_body(x_ref, w_ref, deg_ref, g_ref):
    deg = deg_ref[0, :, 0] + deg_ref[1, :, 0] + 1.0
    dis = lax.rsqrt(deg)
    h = jnp.dot(x_ref[...], w_ref[...], preferred_element_type=jnp.float32)
    g_ref[...] = h * dis[:, None]


def _gw_call(x, W, degp2):
    return pl.pallas_call(
        _gw_body,
        grid=(_NRB,),
        in_specs=[
            pl.BlockSpec((_RB, D), lambda i: (i, 0)),
            pl.BlockSpec((D, D), lambda i: (0, 0)),
            pl.BlockSpec((2, _RB, 16), lambda i: (0, i, 0)),
        ],
        out_specs=pl.BlockSpec((_RB, D), lambda i: (i, 0)),
        out_shape=jax.ShapeDtypeStruct((N_NODES, D), jnp.float32),
    )(x, W, degp2)


def _fin_body(a0_ref, a1_ref, g_ref, deg_ref, b_ref, out_ref):
    deg = deg_ref[0, :, 0] + deg_ref[1, :, 0] + 1.0
    dis = lax.rsqrt(deg)
    acc = a0_ref[...] + a1_ref[...] + g_ref[...]
    out_ref[...] = acc * dis[:, None] + b_ref[...]


def _fin_call(a0, a1, g, degp2, b2):
    return pl.pallas_call(
        _fin_body,
        grid=(_NRB,),
        in_specs=[
            pl.BlockSpec((_RB, D), lambda i: (i, 0)),
            pl.BlockSpec((_RB, D), lambda i: (i, 0)),
            pl.BlockSpec((_RB, D), lambda i: (i, 0)),
            pl.BlockSpec((2, _RB, 16), lambda i: (0, i, 0)),
            pl.BlockSpec((1, D), lambda i: (0, 0)),
        ],
        out_specs=pl.BlockSpec((_RB, D), lambda i: (i, 0)),
        out_shape=jax.ShapeDtypeStruct((N_NODES, D), jnp.float32),
    )(a0, a1, g, degp2, b2)


def kernel(x, adj, W, b):
    src = adj[0]
    dst = adj[1]
    pad = jnp.full((EP - N_EDGES,), PAD_ROW, dtype=jnp.int32)
    srcb = jnp.concatenate([src, pad]).reshape(NW, BPT, BLK)
    dstb = jnp.concatenate([dst, pad]).reshape(NW, BPT, BLK)

    zeros_rt16 = jnp.zeros((RPT, 16), jnp.float32)
    ones_b16 = jnp.ones((BLK, 16), jnp.float32)
    zeros_rtd = jnp.zeros((RPT, D), jnp.float32)

    degp = _deg_call(dstb, zeros_rt16, ones_b16)
    degp2 = jnp.stack([degp[:N_NODES], degp[NT:NT + N_NODES]])

    g = _gw_call(x, W, degp2)
    g_pad = jnp.concatenate([g, jnp.zeros((NT - N_NODES, D), jnp.float32)])

    accp = _agg_call(g_pad, srcb, dstb, zeros_rtd)
    a0 = accp[:N_NODES]
    a1 = accp[NT:NT + N_NODES]

    b2 = b.reshape(1, D)
    return _fin_call(a0, a1, g, degp2, b2)
